# baseline (speedup 1.0000x reference)
import functools

import jax
import jax.numpy as jnp
from jax import lax
from jax.experimental import pallas as pl
from jax.experimental.pallas import tpu as pltpu
from jax.experimental.pallas import tpu_sc as plsc

_B, _C, _P, _T = 8, 64, 12, 1024
_O = 6
_PAIRS = _B * _C
_NC = 2
_NS = 16
_NW = _NC * _NS
_PER_W = _PAIRS // _NW
_L = 16
_H = 2
_HR = _H * _P
_NCHUNK = _O // _H
_TC = _T // _L
_IN = _P * _T
_HE = _H * _IN


def _sc_body(tone_hbm, idx_hbm, out_hbm, val0, val1, idx0, idx1,
             out_a, out_b, sem_in, sem_a, sem_b):
    wid = lax.axis_index("s") * _NC + lax.axis_index("c")
    first = wid * _PER_W
    bufs = ((val0, idx0), (val1, idx1))

    def in_copies(k, slot):
        vv, iv = bufs[slot]
        b = (first + k) // _C
        c = lax.rem(first + k, _C)
        return (
            pltpu.make_async_copy(tone_hbm.at[b, c], vv, sem_in),
            pltpu.make_async_copy(idx_hbm.at[b, c], iv, sem_in),
        )

    def out_copy(buf, sem, k, h):
        b = (first + k) // _C
        c = lax.rem(first + k, _C)
        return pltpu.make_async_copy(
            buf, out_hbm.at[b, c, pl.ds(h * (_HR // 8), _HR // 8)], sem)

    for cp in in_copies(0, 0):
        cp.start()

    def compute_half(slot, h, buf):
        vv, iv = bufs[slot]

        for p in range(_P):
            def t_body(j, cp2, p=p):
                ts = j * _L
                v = vv[p, pl.ds(ts, _L)]
                ix = iv[p, pl.ds(ts, _L)]
                zero = jnp.zeros((_L,), jnp.float32)
                for oo in range(_H):
                    row = oo * _P + p
                    buf[row // 8, row % 8, pl.ds(ts, _L)] = (
                        jnp.where(ix == h * _H + oo, v, zero))
                return cp2

            lax.fori_loop(0, _TC, t_body, 0, unroll=4)

    def one_pair(k, slot, par):
        for cp in in_copies(k, slot):
            cp.wait()

        @pl.when(k + 1 < _PER_W)
        def _prefetch():
            for cp in in_copies(k + 1, 1 - slot):
                cp.start()

        obufs = ((out_a, sem_a), (out_b, sem_b))
        for h in range(_NCHUNK):
            buf, sem = obufs[(par + h) % 2]
            prev_k, prev_h = (k, h - 2) if h >= 2 else (k - 1, h + 1)

            @pl.when(k + (1 if h >= 2 else 0) > 0)
            def _drain():
                out_copy(buf, sem, prev_k, prev_h).wait()

            compute_half(slot, h, buf)
            out_copy(buf, sem, k, h).start()

    def pair_body(j, carry):
        one_pair(2 * j, 0, 0)
        one_pair(2 * j + 1, 1, 1)
        return carry

    lax.fori_loop(0, _PER_W // 2, pair_body, 0)
    out_copy(out_a, sem_a, _PER_W - 1, 1).wait()
    out_copy(out_b, sem_b, _PER_W - 1, 2).wait()


@jax.jit
def _unpool(tone4, idx4):
    mesh = plsc.VectorSubcoreMesh(core_axis_name="c", subcore_axis_name="s")
    return pl.kernel(
        _sc_body,
        mesh=mesh,
        out_type=jax.ShapeDtypeStruct((_B, _C, _O * _P // 8, 8, _T), jnp.float32),
        scratch_types=[
            pltpu.VMEM((_P, _T), jnp.float32),
            pltpu.VMEM((_P, _T), jnp.float32),
            pltpu.VMEM((_P, _T), jnp.int32),
            pltpu.VMEM((_P, _T), jnp.int32),
            pltpu.VMEM((_HR // 8, 8, _T), jnp.float32),
            pltpu.VMEM((_HR // 8, 8, _T), jnp.float32),
            pltpu.SemaphoreType.DMA,
            pltpu.SemaphoreType.DMA,
            pltpu.SemaphoreType.DMA,
        ],
    )(tone4, idx4)


def kernel(tone_out, idx):
    return _unpool(tone_out, idx).reshape(_B, _C, _O * _P, _T)


# speedup vs baseline: 1.5527x; 1.5527x over previous
import functools

import jax
import jax.numpy as jnp
from jax import lax
from jax.experimental import pallas as pl
from jax.experimental.pallas import tpu as pltpu
from jax.experimental.pallas import tpu_sc as plsc

_B, _C, _P, _T = 8, 64, 12, 1024
_O = 6
_PAIRS = _B * _C
_NC = 2
_NS = 16
_NW = _NC * _NS
_PER_W = _PAIRS // _NW
_L = 16
_H = 2
_HR = _H * _P
_NCHUNK = _O // _H
_TC = _T // _L
_IN = _P * _T
_HE = _H * _IN


def _sc_body(tone_hbm, idx_hbm, out_hbm, val0, val1, idx0, idx1,
             out_a, out_b, sem_in, sem_a, sem_b):
    wid = lax.axis_index("s") * _NC + lax.axis_index("c")
    first = wid * _PER_W
    bufs = ((val0, idx0), (val1, idx1))

    def in_copies(k, slot):
        vv, iv = bufs[slot]
        b = (first + k) // _C
        c = lax.rem(first + k, _C)
        return (
            pltpu.make_async_copy(tone_hbm.at[b, c], vv, sem_in),
            pltpu.make_async_copy(idx_hbm.at[b, c], iv, sem_in),
        )

    def out_copy(buf, sem, k, h):
        b = (first + k) // _C
        c = lax.rem(first + k, _C)
        return pltpu.make_async_copy(
            buf, out_hbm.at[b, c, pl.ds(h * (_HR // 8), _HR // 8)], sem)

    for cp in in_copies(0, 0):
        cp.start()

    def compute_half(slot, h, buf):
        vv, iv = bufs[slot]

        for p in range(_P):
            def t_body(j, cp2, p=p):
                ts = j * _L
                v = vv[p, pl.ds(ts, _L)]
                ix = iv[p, pl.ds(ts, _L)]
                zero = jnp.zeros((_L,), jnp.float32)
                for oo in range(_H):
                    row = oo * _P + p
                    buf[row // 8, row % 8, pl.ds(ts, _L)] = (
                        jnp.where(ix == h * _H + oo, v, zero))
                return cp2

            lax.fori_loop(0, _TC, t_body, 0, unroll=8)

    def one_pair(k, slot, par):
        for cp in in_copies(k, slot):
            cp.wait()

        @pl.when(k + 1 < _PER_W)
        def _prefetch():
            for cp in in_copies(k + 1, 1 - slot):
                cp.start()

        obufs = ((out_a, sem_a), (out_b, sem_b))
        for h in range(_NCHUNK):
            buf, sem = obufs[(par + h) % 2]
            prev_k, prev_h = (k, h - 2) if h >= 2 else (k - 1, h + 1)

            @pl.when(k + (1 if h >= 2 else 0) > 0)
            def _drain():
                out_copy(buf, sem, prev_k, prev_h).wait()

            compute_half(slot, h, buf)
            out_copy(buf, sem, k, h).start()

    def pair_body(j, carry):
        one_pair(2 * j, 0, 0)
        one_pair(2 * j + 1, 1, 1)
        return carry

    lax.fori_loop(0, _PER_W // 2, pair_body, 0)
    out_copy(out_a, sem_a, _PER_W - 1, 1).wait()
    out_copy(out_b, sem_b, _PER_W - 1, 2).wait()


@jax.jit
def _unpool(tone4, idx4):
    mesh = plsc.VectorSubcoreMesh(core_axis_name="c", subcore_axis_name="s")
    return pl.kernel(
        _sc_body,
        mesh=mesh,
        out_type=jax.ShapeDtypeStruct((_B, _C, _O * _P // 8, 8, _T), jnp.float32),
        scratch_types=[
            pltpu.VMEM((_P, _T), jnp.float32),
            pltpu.VMEM((_P, _T), jnp.float32),
            pltpu.VMEM((_P, _T), jnp.int32),
            pltpu.VMEM((_P, _T), jnp.int32),
            pltpu.VMEM((_HR // 8, 8, _T), jnp.float32),
            pltpu.VMEM((_HR // 8, 8, _T), jnp.float32),
            pltpu.SemaphoreType.DMA,
            pltpu.SemaphoreType.DMA,
            pltpu.SemaphoreType.DMA,
        ],
    )(tone4, idx4)


def kernel(tone_out, idx):
    return _unpool(tone_out, idx).reshape(_B, _C, _O * _P, _T)
